# baseline (speedup 1.0000x reference)
import functools

import jax
import jax.numpy as jnp
from jax import lax
from jax.experimental import pallas as pl
from jax.experimental.pallas import tpu as pltpu
from jax.experimental.pallas import tpu_sc as plsc

_N = 10000
_E = 320000
_H = 128
_NS = 16
_ROWS_A = 640
_ROWS_LAST = _N - 15 * _ROWS_A
_CHUNK = 128
_NFULL = 156
_EDGES_PER_TILE = _NFULL * _CHUNK
_NEXTRA = (_E - _NS * _EDGES_PER_TILE) // _CHUNK
_GRP = 12


def _sc_body(x_hbm, ei_hbm, out0_hbm, out1_hbm, acc,
             ib0, ib1, ib2, ib3, rb0, rb1, rb2,
             i0, i1, i2, i3, g0, g1, g2, s0, s1, s2):
    c = lax.axis_index("c")
    s = lax.axis_index("s")
    ib = [ib0, ib1, ib2, ib3]
    isem = [i0, i1, i2, i3]
    rb = [rb0, rb1, rb2]
    gsem = [g0, g1, g2]
    ssem = [s0, s1, s2]

    r0 = s * _ROWS_A
    ebase = s * _EDGES_PER_TILE

    def seed_acc():
        @pl.when(s < _NS - 1)
        def _():
            pltpu.sync_copy(x_hbm.at[pl.ds(r0, _ROWS_A)],
                            acc.at[pl.ds(r0, _ROWS_A)])

        @pl.when(s == _NS - 1)
        def _():
            pltpu.sync_copy(x_hbm.at[pl.ds(15 * _ROWS_A, _ROWS_LAST)],
                            acc.at[pl.ds(15 * _ROWS_A, _ROWS_LAST)])

    def run_direction(srow, drow):
        def issue_idx(j, b):
            q = b % 4
            off = ebase + j * _CHUNK
            pltpu.async_copy(ei_hbm.at[:, pl.ds(off, _CHUNK)], ib[q], isem[q])

        def wait_idx(b):
            q = b % 4
            pltpu.make_async_copy(ei_hbm.at[:, pl.ds(0, _CHUNK)], ib[q],
                                  isem[q]).wait()

        def issue_gather(b):
            pltpu.async_copy(x_hbm.at[ib[b % 4].at[srow]], rb[b % 3],
                             gsem[b % 3])

        def wait_gather(b):
            pltpu.make_async_copy(x_hbm.at[pl.ds(0, _CHUNK)], rb[b % 3],
                                  gsem[b % 3]).wait()

        def issue_scatter(b):
            pltpu.async_copy(rb[b % 3], acc.at[ib[b % 4].at[drow]],
                             ssem[b % 3], add=True)

        def wait_scatter(b):
            pltpu.make_async_copy(rb[b % 3], acc.at[pl.ds(0, _CHUNK)],
                                  ssem[b % 3]).wait()

        def chunk_step(j, b, has_g2, has_i4, has_ws=True):
            if has_g2:
                wait_idx(b + 2)
                if has_ws:
                    wait_scatter(b + 2)
                issue_gather(b + 2)
            wait_gather(b)
            issue_scatter(b)
            if has_i4:
                issue_idx(j + 4, b + 4)

        issue_idx(0, 0)
        issue_idx(1, 1)
        issue_idx(2, 2)
        issue_idx(3, 3)
        wait_idx(0)
        issue_gather(0)
        wait_idx(1)
        issue_gather(1)
        seed_acc()
        plsc.subcore_barrier()

        for j in range(_GRP):
            chunk_step(j, j, True, True, has_ws=(j >= 1))

        def group_body(k, carry):
            for b in range(_GRP):
                chunk_step(k * _GRP + b, b, True, True)
            return carry

        lax.fori_loop(1, (_NFULL // _GRP) - 1, group_body, 0)
        for j in range(_NFULL - _GRP, _NFULL):
            chunk_step(j, j % _GRP, j + 2 < _NFULL, j + 4 < _NFULL)
        wait_scatter(0)
        wait_scatter(1)
        wait_scatter(2)

        @pl.when(s < _NEXTRA)
        def _():
            xoff = _NS * _EDGES_PER_TILE + s * _CHUNK
            pltpu.sync_copy(ei_hbm.at[:, pl.ds(xoff, _CHUNK)], ib[0])
            pltpu.async_copy(x_hbm.at[ib[0].at[srow]], rb[0], gsem[0])
            pltpu.make_async_copy(x_hbm.at[pl.ds(0, _CHUNK)], rb[0],
                                  gsem[0]).wait()
            pltpu.sync_copy(rb[0], acc.at[ib[0].at[drow]], add=True)

    @pl.when(c == 0)
    def _():
        run_direction(0, 1)

    @pl.when(c == 1)
    def _():
        run_direction(1, 0)

    plsc.subcore_barrier()

    def writeout(out_hbm):
        @pl.when(s < _NS - 1)
        def _():
            pltpu.sync_copy(acc.at[pl.ds(r0, _ROWS_A)],
                            out_hbm.at[pl.ds(r0, _ROWS_A)])

        @pl.when(s == _NS - 1)
        def _():
            pltpu.sync_copy(acc.at[pl.ds(15 * _ROWS_A, _ROWS_LAST)],
                            out_hbm.at[pl.ds(15 * _ROWS_A, _ROWS_LAST)])

    @pl.when(c == 0)
    def _():
        writeout(out0_hbm)

    @pl.when(c == 1)
    def _():
        writeout(out1_hbm)


_sc_agg = functools.partial(
    pl.kernel,
    out_type=[jax.ShapeDtypeStruct((_N, _H), jnp.float32),
              jax.ShapeDtypeStruct((_N, _H), jnp.float32)],
    mesh=plsc.VectorSubcoreMesh(core_axis_name="c", subcore_axis_name="s"),
    scratch_types=[
        pltpu.VMEM_SHARED((_N, _H), jnp.float32),
        pltpu.VMEM((2, _CHUNK), jnp.int32),
        pltpu.VMEM((2, _CHUNK), jnp.int32),
        pltpu.VMEM((2, _CHUNK), jnp.int32),
        pltpu.VMEM((2, _CHUNK), jnp.int32),
        pltpu.VMEM((_CHUNK, _H), jnp.float32),
        pltpu.VMEM((_CHUNK, _H), jnp.float32),
        pltpu.VMEM((_CHUNK, _H), jnp.float32),
        pltpu.SemaphoreType.DMA,
        pltpu.SemaphoreType.DMA,
        pltpu.SemaphoreType.DMA,
        pltpu.SemaphoreType.DMA,
        pltpu.SemaphoreType.DMA,
        pltpu.SemaphoreType.DMA,
        pltpu.SemaphoreType.DMA,
        pltpu.SemaphoreType.DMA,
        pltpu.SemaphoreType.DMA,
        pltpu.SemaphoreType.DMA,
    ],
)(_sc_body)


_BLK = 2000


def _matT(a, w):
    return lax.dot_general(a, w, (((1,), (1,)), ((), ())),
                           preferred_element_type=jnp.float32)


def _dense_body(h1_ref, h2_ref, W1_ref, b1_ref, g1_ref, be1_ref,
                W2_ref, b2_ref, g2_ref, be2_ref,
                Wl1a_ref, Wl1b_ref, bl1_ref, Wl2_ref, bl2_ref, out_ref):
    def resblock(h, W, b, g, be):
        z = _matT(h, W) + b
        mu = jnp.mean(z, axis=-1, keepdims=True)
        var = jnp.mean((z - mu) * (z - mu), axis=-1, keepdims=True)
        ln = (z - mu) * lax.rsqrt(var + 1e-5) * g + be
        return h + jnp.maximum(ln, 0.0)

    r1 = resblock(h1_ref[:], W1_ref[:], b1_ref[:], g1_ref[:], be1_ref[:])
    r2 = resblock(h2_ref[:], W2_ref[:], b2_ref[:], g2_ref[:], be2_ref[:])
    hmid = jnp.maximum(
        _matT(r1, Wl1a_ref[:]) + _matT(r2, Wl1b_ref[:]) + bl1_ref[:], 0.0)
    out_ref[:] = jnp.maximum(_matT(hmid, Wl2_ref[:]) + bl2_ref[:], 0.0)


def _row_spec(nrows, ncols):
    return pl.BlockSpec((nrows, ncols), lambda i: (i, 0))


def _full_spec(nrows, ncols):
    return pl.BlockSpec((nrows, ncols), lambda i: (0, 0))


_dense_call = pl.pallas_call(
    _dense_body,
    grid=(_N // _BLK,),
    in_specs=[
        _row_spec(_BLK, _H), _row_spec(_BLK, _H),
        _full_spec(_H, _H), _full_spec(1, _H), _full_spec(1, _H), _full_spec(1, _H),
        _full_spec(_H, _H), _full_spec(1, _H), _full_spec(1, _H), _full_spec(1, _H),
        _full_spec(2 * _H, _H), _full_spec(2 * _H, _H), _full_spec(1, 2 * _H),
        _full_spec(_H, 2 * _H), _full_spec(1, _H),
    ],
    out_specs=_row_spec(_BLK, _H),
    out_shape=jax.ShapeDtypeStruct((_N, _H), jnp.float32),
)


@jax.jit
def _impl(x, ei, W1, b1, g1, be1, W2, b2, g2, be2, Wl1, bl1, Wl2, bl2):
    h1, h2 = _sc_agg(x, ei)
    return _dense_call(
        h1, h2,
        W1, b1[None, :], g1[None, :], be1[None, :],
        W2, b2[None, :], g2[None, :], be2[None, :],
        Wl1[:, :_H], Wl1[:, _H:], bl1[None, :],
        Wl2, bl2[None, :],
    )


def kernel(x, ei, W1, b1, g1, be1, W2, b2, g2, be2, Wl1, bl1, Wl2, bl2):
    return _impl(x, ei, W1, b1, g1, be1, W2, b2, g2, be2, Wl1, bl1, Wl2, bl2)


# speedup vs baseline: 1.0034x; 1.0034x over previous
import functools

import jax
import jax.numpy as jnp
from jax import lax
from jax.experimental import pallas as pl
from jax.experimental.pallas import tpu as pltpu
from jax.experimental.pallas import tpu_sc as plsc

_N = 10000
_E = 320000
_H = 128
_NS = 16
_ROWS_A = 640
_ROWS_LAST = _N - 15 * _ROWS_A
_CHUNK = 128
_NFULL = 156
_EDGES_PER_TILE = _NFULL * _CHUNK
_NEXTRA = (_E - _NS * _EDGES_PER_TILE) // _CHUNK
_GRP = 12


def _sc_body(x_hbm, ei_hbm, out0_hbm, out1_hbm, acc,
             ib0, ib1, ib2, ib3, rb0, rb1, rb2,
             i0, i1, i2, i3, g0, g1, g2):
    c = lax.axis_index("c")
    s = lax.axis_index("s")
    ib = [ib0, ib1, ib2, ib3]
    isem = [i0, i1, i2, i3]
    rb = [rb0, rb1, rb2]
    gsem = [g0, g1, g2]

    r0 = s * _ROWS_A
    ebase = s * _EDGES_PER_TILE

    def seed_acc():
        @pl.when(s < _NS - 1)
        def _():
            pltpu.sync_copy(x_hbm.at[pl.ds(r0, _ROWS_A)],
                            acc.at[pl.ds(r0, _ROWS_A)])

        @pl.when(s == _NS - 1)
        def _():
            pltpu.sync_copy(x_hbm.at[pl.ds(15 * _ROWS_A, _ROWS_LAST)],
                            acc.at[pl.ds(15 * _ROWS_A, _ROWS_LAST)])

    def run_direction(srow, drow):
        def issue_idx(j, b):
            q = b % 4
            off = ebase + j * _CHUNK
            pltpu.async_copy(ei_hbm.at[:, pl.ds(off, _CHUNK)], ib[q], isem[q])

        def wait_idx(b):
            q = b % 4
            pltpu.make_async_copy(ei_hbm.at[:, pl.ds(0, _CHUNK)], ib[q],
                                  isem[q]).wait()

        def issue_gather(b):
            pltpu.async_copy(x_hbm.at[ib[b % 4].at[srow]], rb[b % 3],
                             gsem[b % 3])

        def wait_gather(b):
            pltpu.make_async_copy(x_hbm.at[pl.ds(0, _CHUNK)], rb[b % 3],
                                  gsem[b % 3]).wait()

        def scatter(b):
            pltpu.sync_copy(rb[b % 3], acc.at[ib[b % 4].at[drow]], add=True)

        def chunk_step(j, b, has_g2, has_i4):
            if has_g2:
                wait_idx(b + 2)
                issue_gather(b + 2)
            wait_gather(b)
            scatter(b)
            if has_i4:
                issue_idx(j + 4, b + 4)

        issue_idx(0, 0)
        issue_idx(1, 1)
        issue_idx(2, 2)
        issue_idx(3, 3)
        wait_idx(0)
        issue_gather(0)
        wait_idx(1)
        issue_gather(1)
        seed_acc()
        plsc.subcore_barrier()

        def group_body(k, carry):
            for b in range(_GRP):
                chunk_step(k * _GRP + b, b, True, True)
            return carry

        lax.fori_loop(0, (_NFULL // _GRP) - 1, group_body, 0)
        for j in range(_NFULL - _GRP, _NFULL):
            chunk_step(j, j % _GRP, j + 2 < _NFULL, j + 4 < _NFULL)

        @pl.when(s < _NEXTRA)
        def _():
            xoff = _NS * _EDGES_PER_TILE + s * _CHUNK
            pltpu.sync_copy(ei_hbm.at[:, pl.ds(xoff, _CHUNK)], ib[0])
            pltpu.async_copy(x_hbm.at[ib[0].at[srow]], rb[0], gsem[0])
            pltpu.make_async_copy(x_hbm.at[pl.ds(0, _CHUNK)], rb[0],
                                  gsem[0]).wait()
            pltpu.sync_copy(rb[0], acc.at[ib[0].at[drow]], add=True)

    @pl.when(c == 0)
    def _():
        run_direction(0, 1)

    @pl.when(c == 1)
    def _():
        run_direction(1, 0)

    plsc.subcore_barrier()

    def writeout(out_hbm):
        @pl.when(s < _NS - 1)
        def _():
            pltpu.sync_copy(acc.at[pl.ds(r0, _ROWS_A)],
                            out_hbm.at[pl.ds(r0, _ROWS_A)])

        @pl.when(s == _NS - 1)
        def _():
            pltpu.sync_copy(acc.at[pl.ds(15 * _ROWS_A, _ROWS_LAST)],
                            out_hbm.at[pl.ds(15 * _ROWS_A, _ROWS_LAST)])

    @pl.when(c == 0)
    def _():
        writeout(out0_hbm)

    @pl.when(c == 1)
    def _():
        writeout(out1_hbm)


_sc_agg = functools.partial(
    pl.kernel,
    out_type=[jax.ShapeDtypeStruct((_N, _H), jnp.float32),
              jax.ShapeDtypeStruct((_N, _H), jnp.float32)],
    mesh=plsc.VectorSubcoreMesh(core_axis_name="c", subcore_axis_name="s"),
    scratch_types=[
        pltpu.VMEM_SHARED((_N, _H), jnp.float32),
        pltpu.VMEM((2, _CHUNK), jnp.int32),
        pltpu.VMEM((2, _CHUNK), jnp.int32),
        pltpu.VMEM((2, _CHUNK), jnp.int32),
        pltpu.VMEM((2, _CHUNK), jnp.int32),
        pltpu.VMEM((_CHUNK, _H), jnp.float32),
        pltpu.VMEM((_CHUNK, _H), jnp.float32),
        pltpu.VMEM((_CHUNK, _H), jnp.float32),
        pltpu.SemaphoreType.DMA,
        pltpu.SemaphoreType.DMA,
        pltpu.SemaphoreType.DMA,
        pltpu.SemaphoreType.DMA,
        pltpu.SemaphoreType.DMA,
        pltpu.SemaphoreType.DMA,
        pltpu.SemaphoreType.DMA,
    ],
)(_sc_body)


_BLK = 2000


def _matT(a, w):
    return lax.dot_general(a, w, (((1,), (1,)), ((), ())),
                           preferred_element_type=jnp.float32)


def _dense_body(h1_ref, h2_ref, W1_ref, b1_ref, g1_ref, be1_ref,
                W2_ref, b2_ref, g2_ref, be2_ref,
                Wl1a_ref, Wl1b_ref, bl1_ref, Wl2_ref, bl2_ref, out_ref):
    def resblock(h, W, b, g, be):
        z = _matT(h, W) + b
        mu = jnp.mean(z, axis=-1, keepdims=True)
        var = jnp.mean((z - mu) * (z - mu), axis=-1, keepdims=True)
        ln = (z - mu) * lax.rsqrt(var + 1e-5) * g + be
        return h + jnp.maximum(ln, 0.0)

    r1 = resblock(h1_ref[:], W1_ref[:], b1_ref[:], g1_ref[:], be1_ref[:])
    r2 = resblock(h2_ref[:], W2_ref[:], b2_ref[:], g2_ref[:], be2_ref[:])
    hmid = jnp.maximum(
        _matT(r1, Wl1a_ref[:]) + _matT(r2, Wl1b_ref[:]) + bl1_ref[:], 0.0)
    out_ref[:] = jnp.maximum(_matT(hmid, Wl2_ref[:]) + bl2_ref[:], 0.0)


def _row_spec(nrows, ncols):
    return pl.BlockSpec((nrows, ncols), lambda i: (i, 0))


def _full_spec(nrows, ncols):
    return pl.BlockSpec((nrows, ncols), lambda i: (0, 0))


_dense_call = pl.pallas_call(
    _dense_body,
    grid=(_N // _BLK,),
    in_specs=[
        _row_spec(_BLK, _H), _row_spec(_BLK, _H),
        _full_spec(_H, _H), _full_spec(1, _H), _full_spec(1, _H), _full_spec(1, _H),
        _full_spec(_H, _H), _full_spec(1, _H), _full_spec(1, _H), _full_spec(1, _H),
        _full_spec(2 * _H, _H), _full_spec(2 * _H, _H), _full_spec(1, 2 * _H),
        _full_spec(_H, 2 * _H), _full_spec(1, _H),
    ],
    out_specs=_row_spec(_BLK, _H),
    out_shape=jax.ShapeDtypeStruct((_N, _H), jnp.float32),
)


@jax.jit
def _impl(x, ei, W1, b1, g1, be1, W2, b2, g2, be2, Wl1, bl1, Wl2, bl2):
    h1, h2 = _sc_agg(x, ei)
    return _dense_call(
        h1, h2,
        W1, b1[None, :], g1[None, :], be1[None, :],
        W2, b2[None, :], g2[None, :], be2[None, :],
        Wl1[:, :_H], Wl1[:, _H:], bl1[None, :],
        Wl2, bl2[None, :],
    )


def kernel(x, ei, W1, b1, g1, be1, W2, b2, g2, be2, Wl1, bl1, Wl2, bl2):
    return _impl(x, ei, W1, b1, g1, be1, W2, b2, g2, be2, Wl1, bl1, Wl2, bl2)
